# Initial kernel scaffold; baseline (speedup 1.0000x reference)
import jax
import jax.numpy as jnp
from jax.experimental import pallas as pl


def kernel(user_indices, item_indices, timestamps, interaction_features, memory, last_update, item_table, W1, b1, W2, b2, W_ih, W_hh, b_ih, b_hh):
    raise NotImplementedError("write your pallas kernel here")


# speedup vs baseline: 4.4763x; 4.4763x over previous
import functools

import jax
import jax.numpy as jnp
from jax import lax
from jax.experimental import pallas as pl
from jax.experimental.pallas import tpu as pltpu
from jax.experimental.pallas import tpu_sc as plsc


@functools.lru_cache(maxsize=None)
def _make_gather(num_users, num_items, d, batch):
  nc, ns = 2, 16
  nw = nc * ns
  bpw = batch // nw
  nchunk = bpw // 128
  mesh = plsc.VectorSubcoreMesh(core_axis_name="c", subcore_axis_name="s")

  @functools.partial(
      pl.kernel,
      out_type=(
          jax.ShapeDtypeStruct((batch, d), jnp.float32),
          jax.ShapeDtypeStruct((batch, d), jnp.float32),
      ),
      mesh=mesh,
      scratch_types=[
          pltpu.VMEM((nchunk, 128), jnp.int32),
          pltpu.VMEM((nchunk, 128), jnp.int32),
          pltpu.VMEM((128, d), jnp.float32),
          pltpu.VMEM((128, d), jnp.float32),
          pltpu.SemaphoreType.DMA,
          pltpu.SemaphoreType.DMA,
      ],
  )
  def gather(mem_hbm, item_hbm, uidx_hbm, iidx_hbm, umem_out, iemb_out,
             uidx_v, iidx_v, rows_u, rows_i, sem_u, sem_i):
    c = lax.axis_index("c")
    s = lax.axis_index("s")
    wid = s * nc + c
    base = wid * nchunk
    pltpu.sync_copy(uidx_hbm.at[pl.ds(base, nchunk)], uidx_v)
    pltpu.sync_copy(iidx_hbm.at[pl.ds(base, nchunk)], iidx_v)
    for j in range(nchunk):
      r = (base + j) * 128
      cp_u = pltpu.async_copy(mem_hbm.at[uidx_v.at[j]], rows_u, sem_u)
      cp_i = pltpu.async_copy(item_hbm.at[iidx_v.at[j]], rows_i, sem_i)
      cp_u.wait()
      pltpu.sync_copy(rows_u, umem_out.at[pl.ds(r, 128)])
      cp_i.wait()
      pltpu.sync_copy(rows_i, iemb_out.at[pl.ds(r, 128)])

  return gather


@functools.lru_cache(maxsize=None)
def _make_dense(batch, d, m, num_users, bm):
  grid = (batch // bm,)

  def body(um, ie, ft, w1u, w1i, w1f, b1, w2, b2, wih, whh, bih, bhh,
           mem_in, nm_out, mem_out):
    del mem_in, mem_out
    h = jnp.dot(um[...], w1u[...], preferred_element_type=jnp.float32)
    h = h + jnp.dot(ie[...], w1i[...], preferred_element_type=jnp.float32)
    h = h + jnp.dot(ft[...], w1f[...], preferred_element_type=jnp.float32)
    h = jnp.maximum(h + b1[...], 0.0)
    msg = jnp.dot(h, w2[...], preferred_element_type=jnp.float32) + b2[...]
    gi = jnp.dot(msg, wih[...], preferred_element_type=jnp.float32) + bih[...]
    gh = jnp.dot(um[...], whh[...], preferred_element_type=jnp.float32) + bhh[...]
    r = jax.nn.sigmoid(gi[:, :d] + gh[:, :d])
    z = jax.nn.sigmoid(gi[:, d:2 * d] + gh[:, d:2 * d])
    n = jnp.tanh(gi[:, 2 * d:] + r * gh[:, 2 * d:])
    nm_out[...] = (1.0 - z) * n + z * um[...]

  bspec = lambda shape: pl.BlockSpec(shape, lambda i: (0, 0))
  return pl.pallas_call(
      body,
      grid=grid,
      in_specs=[
          pl.BlockSpec((bm, d), lambda i: (i, 0)),
          pl.BlockSpec((bm, d), lambda i: (i, 0)),
          pl.BlockSpec((bm, m), lambda i: (i, 0)),
          bspec((d, m)),
          bspec((d, m)),
          bspec((m, m)),
          bspec((1, m)),
          bspec((m, m)),
          bspec((1, m)),
          bspec((m, 3 * d)),
          bspec((d, 3 * d)),
          bspec((1, 3 * d)),
          bspec((1, 3 * d)),
          pl.BlockSpec(memory_space=pl.ANY),
      ],
      out_specs=[
          pl.BlockSpec((bm, d), lambda i: (i, 0)),
          pl.BlockSpec(memory_space=pl.ANY),
      ],
      out_shape=[
          jax.ShapeDtypeStruct((batch, d), jnp.float32),
          jax.ShapeDtypeStruct((num_users, d), jnp.float32),
      ],
      input_output_aliases={13: 1},
      compiler_params=pltpu.CompilerParams(
          dimension_semantics=("arbitrary",)),
  )


@functools.lru_cache(maxsize=None)
def _make_scatter(num_users, d, batch):
  ns = 16
  bpw = batch // ns
  nchunk = bpw // 128
  lu_chunk = 5000
  n_lu = num_users // lu_chunk
  lu_per = (n_lu + ns - 1) // ns
  tag_size = 1048704
  tag_per = tag_size // ns
  dummy = num_users
  mesh = plsc.VectorSubcoreMesh(
      core_axis_name="c", subcore_axis_name="s", num_cores=1)

  @functools.partial(
      pl.kernel,
      out_type=jax.ShapeDtypeStruct((num_users,), jnp.float32),
      mesh=mesh,
      scratch_types=[
          pltpu.VMEM((nchunk, 128), jnp.int32),
          pltpu.VMEM((nchunk, 128), jnp.int32),
          pltpu.VMEM((nchunk, 128), jnp.int32),
          pltpu.VMEM((nchunk, 128), jnp.int32),
          pltpu.VMEM((8192,), jnp.int32),
          pltpu.VMEM((128, d), jnp.float32),
          pltpu.VMEM((128,), jnp.float32),
          pltpu.VMEM((lu_chunk,), jnp.float32),
          pltpu.VMEM_SHARED((tag_size,), jnp.int32),
          pltpu.SemaphoreType.DMA,
          pltpu.SemaphoreType.DMA,
      ],
      compiler_params=pltpu.CompilerParams(has_side_effects=True),
  )
  def scatter(uidx_hbm, nm_hbm, ts_hbm, lu_hbm, memc_hbm, lu_out,
              uidx_v, e_v, widx_v, srcs_v, zer_v, rows_v, ts_v, lubuf,
              tag_sh, sem, sem2):
    w = lax.axis_index("s")
    base = w * nchunk
    pltpu.sync_copy(uidx_hbm.at[pl.ds(base, nchunk)], uidx_v)
    lanes = lax.iota(jnp.int32, 16)
    for j in range(nchunk):
      for k in range(8):
        e_v[j, pl.ds(k * 16, 16)] = w * bpw + j * 128 + k * 16 + lanes
    for k in range(256):
      zer_v[pl.ds(k * 16, 16)] = jnp.zeros((16,), jnp.int32)
    tbase = w * tag_per
    for t in range(16):
      pltpu.sync_copy(zer_v.at[pl.ds(0, 4096)],
                      tag_sh.at[pl.ds(tbase + t * 4096, 4096)])
    pltpu.sync_copy(zer_v.at[pl.ds(0, 8)],
                    tag_sh.at[pl.ds(tbase + 16 * 4096, 8)])
    for t in range(lu_per):
      ci = w + t * ns

      @pl.when(ci < n_lu)
      def _():
        pltpu.sync_copy(lu_hbm.at[pl.ds(ci * lu_chunk, lu_chunk)], lubuf)
        pltpu.sync_copy(lubuf, lu_out.at[pl.ds(ci * lu_chunk, lu_chunk)])

    plsc.subcore_barrier()

    def bit_round(t, carry):
      b = 13 - t
      for j in range(nchunk):
        pltpu.async_copy(tag_sh.at[uidx_v.at[j]], widx_v.at[j], sem).wait()
      for j in range(nchunk):
        for k in range(8):
          sl = pl.ds(k * 16, 16)
          e = e_v[j, sl]
          cur = widx_v[j, sl]
          alive = jnp.logical_and((e >> (b + 1)) == (cur >> (b + 1)),
                                  ((e >> b) & 1) == 1)
          srcs_v[j, sl] = (e >> b) << b
          widx_v[j, sl] = jnp.where(alive, uidx_v[j, sl], dummy)
      for j in range(nchunk):
        pltpu.async_copy(srcs_v.at[j], tag_sh.at[widx_v.at[j]], sem).wait()
      plsc.subcore_barrier()
      return carry

    lax.fori_loop(0, 14, bit_round, jnp.int32(0))

    for j in range(nchunk):
      pltpu.async_copy(tag_sh.at[uidx_v.at[j]], srcs_v.at[j], sem).wait()
      pltpu.async_copy(nm_hbm.at[srcs_v.at[j]], rows_v, sem).wait()
      pltpu.async_copy(rows_v, memc_hbm.at[uidx_v.at[j]], sem).wait()
      pltpu.async_copy(ts_hbm.at[srcs_v.at[j]], ts_v, sem2).wait()
      pltpu.async_copy(ts_v, lu_out.at[uidx_v.at[j]], sem2).wait()

  return scatter


def kernel(user_indices, item_indices, timestamps, interaction_features,
           memory, last_update, item_table, W1, b1, W2, b2, W_ih, W_hh,
           b_ih, b_hh):
  batch = user_indices.shape[0]
  num_users, d = memory.shape
  num_items = item_table.shape[0]
  m = interaction_features.shape[1]

  uidx2 = user_indices.reshape(batch // 128, 128)
  iidx2 = item_indices.reshape(batch // 128, 128)

  user_mem, item_emb = _make_gather(num_users, num_items, d, batch)(
      memory, item_table, uidx2, iidx2)

  w1t = W1.T
  w1u, w1i, w1f = w1t[:d], w1t[d:2 * d], w1t[2 * d:]
  new_mem, mem_c = _make_dense(batch, d, m, num_users, 2048)(
      user_mem, item_emb, interaction_features,
      w1u, w1i, w1f, b1.reshape(1, m), W2.T, b2.reshape(1, m),
      W_ih.T, W_hh.T, b_ih.reshape(1, 3 * d), b_hh.reshape(1, 3 * d),
      memory)

  lu_out = _make_scatter(num_users, d, batch)(
      uidx2, new_mem, timestamps, last_update, mem_c)

  return new_mem, mem_c, lu_out


# speedup vs baseline: 4.6577x; 1.0405x over previous
import functools

import jax
import jax.numpy as jnp
from jax import lax
from jax.experimental import pallas as pl
from jax.experimental.pallas import tpu as pltpu
from jax.experimental.pallas import tpu_sc as plsc


@functools.lru_cache(maxsize=None)
def _make_gather(num_users, num_items, d, batch):
  nc, ns = 2, 16
  nw = nc * ns
  bpw = batch // nw
  nchunk = bpw // 128
  mesh = plsc.VectorSubcoreMesh(core_axis_name="c", subcore_axis_name="s")

  @functools.partial(
      pl.kernel,
      out_type=(
          jax.ShapeDtypeStruct((batch, d), jnp.float32),
          jax.ShapeDtypeStruct((batch, d), jnp.float32),
      ),
      mesh=mesh,
      scratch_types=[
          pltpu.VMEM((nchunk, 128), jnp.int32),
          pltpu.VMEM((nchunk, 128), jnp.int32),
          pltpu.VMEM((128, d), jnp.float32),
          pltpu.VMEM((128, d), jnp.float32),
          pltpu.SemaphoreType.DMA,
          pltpu.SemaphoreType.DMA,
      ],
  )
  def gather(mem_hbm, item_hbm, uidx_hbm, iidx_hbm, umem_out, iemb_out,
             uidx_v, iidx_v, rows_u, rows_i, sem_u, sem_i):
    c = lax.axis_index("c")
    s = lax.axis_index("s")
    wid = s * nc + c
    base = wid * nchunk
    pltpu.sync_copy(uidx_hbm.at[pl.ds(base, nchunk)], uidx_v)
    pltpu.sync_copy(iidx_hbm.at[pl.ds(base, nchunk)], iidx_v)
    for j in range(nchunk):
      r = (base + j) * 128
      cp_u = pltpu.async_copy(mem_hbm.at[uidx_v.at[j]], rows_u, sem_u)
      cp_i = pltpu.async_copy(item_hbm.at[iidx_v.at[j]], rows_i, sem_i)
      cp_u.wait()
      pltpu.sync_copy(rows_u, umem_out.at[pl.ds(r, 128)])
      cp_i.wait()
      pltpu.sync_copy(rows_i, iemb_out.at[pl.ds(r, 128)])

  return gather


@functools.lru_cache(maxsize=None)
def _make_dense(batch, d, m, num_users, bm):
  grid = (batch // bm,)

  def body(um, ie, ft, w1u, w1i, w1f, b1, w2, b2, wih, whh, bih, bhh,
           mem_in, nm_out, mem_out):
    del mem_in, mem_out
    h = jnp.dot(um[...], w1u[...], preferred_element_type=jnp.float32)
    h = h + jnp.dot(ie[...], w1i[...], preferred_element_type=jnp.float32)
    h = h + jnp.dot(ft[...], w1f[...], preferred_element_type=jnp.float32)
    h = jnp.maximum(h + b1[...], 0.0)
    msg = jnp.dot(h, w2[...], preferred_element_type=jnp.float32) + b2[...]
    gi = jnp.dot(msg, wih[...], preferred_element_type=jnp.float32) + bih[...]
    gh = jnp.dot(um[...], whh[...], preferred_element_type=jnp.float32) + bhh[...]
    r = jax.nn.sigmoid(gi[:, :d] + gh[:, :d])
    z = jax.nn.sigmoid(gi[:, d:2 * d] + gh[:, d:2 * d])
    n = jnp.tanh(gi[:, 2 * d:] + r * gh[:, 2 * d:])
    nm_out[...] = (1.0 - z) * n + z * um[...]

  bspec = lambda shape: pl.BlockSpec(shape, lambda i: (0, 0))
  return pl.pallas_call(
      body,
      grid=grid,
      in_specs=[
          pl.BlockSpec((bm, d), lambda i: (i, 0)),
          pl.BlockSpec((bm, d), lambda i: (i, 0)),
          pl.BlockSpec((bm, m), lambda i: (i, 0)),
          bspec((d, m)),
          bspec((d, m)),
          bspec((m, m)),
          bspec((1, m)),
          bspec((m, m)),
          bspec((1, m)),
          bspec((m, 3 * d)),
          bspec((d, 3 * d)),
          bspec((1, 3 * d)),
          bspec((1, 3 * d)),
          pl.BlockSpec(memory_space=pl.ANY),
      ],
      out_specs=[
          pl.BlockSpec((bm, d), lambda i: (i, 0)),
          pl.BlockSpec(memory_space=pl.ANY),
      ],
      out_shape=[
          jax.ShapeDtypeStruct((batch, d), jnp.float32),
          jax.ShapeDtypeStruct((num_users, d), jnp.float32),
      ],
      input_output_aliases={13: 1},
      compiler_params=pltpu.CompilerParams(
          dimension_semantics=("arbitrary",)),
  )


@functools.lru_cache(maxsize=None)
def _make_scatter(num_users, d, batch):
  ns = 16
  bpw = batch // ns
  nchunk = bpw // 128
  lu_chunk = 20000
  n_lu = num_users // lu_chunk
  lu_per = (n_lu + ns - 1) // ns
  tag_size = 1048704
  tag_per = tag_size // ns
  dummy = num_users
  mesh = plsc.VectorSubcoreMesh(
      core_axis_name="c", subcore_axis_name="s", num_cores=1)

  @functools.partial(
      pl.kernel,
      out_type=jax.ShapeDtypeStruct((num_users,), jnp.float32),
      mesh=mesh,
      scratch_types=[
          pltpu.VMEM((nchunk, 128), jnp.int32),
          pltpu.VMEM((nchunk, 128), jnp.int32),
          pltpu.VMEM((nchunk, 128), jnp.int32),
          pltpu.VMEM((nchunk, 128), jnp.int32),
          pltpu.VMEM((4096,), jnp.int32),
          pltpu.VMEM((128, d), jnp.float32),
          pltpu.VMEM((128, d), jnp.float32),
          pltpu.VMEM((nchunk, 128), jnp.float32),
          pltpu.VMEM((lu_chunk,), jnp.float32),
          pltpu.VMEM_SHARED((tag_size,), jnp.int32),
          pltpu.SemaphoreType.DMA,
          pltpu.SemaphoreType.DMA,
          pltpu.SemaphoreType.DMA,
      ],
      compiler_params=pltpu.CompilerParams(has_side_effects=True),
  )
  def scatter(uidx_hbm, nm_hbm, ts_hbm, lu_hbm, memc_hbm, lu_out,
              uidx_v, e_v, widx_v, srcs_v, zer_v, rows_a, rows_b, tsb_v,
              lubuf, tag_sh, sem, sem2, sem3):
    w = lax.axis_index("s")
    base = w * nchunk
    pltpu.sync_copy(uidx_hbm.at[pl.ds(base, nchunk)], uidx_v)
    lanes = lax.iota(jnp.int32, 16)
    for j in range(nchunk):
      for k in range(8):
        e_v[j, pl.ds(k * 16, 16)] = w * bpw + j * 128 + k * 16 + lanes
    for k in range(256):
      zer_v[pl.ds(k * 16, 16)] = jnp.zeros((16,), jnp.int32)
    tbase = w * tag_per
    zcps = [pltpu.async_copy(zer_v, tag_sh.at[pl.ds(tbase + t * 4096, 4096)],
                             sem) for t in range(16)]
    zcps.append(pltpu.async_copy(zer_v.at[pl.ds(0, 8)],
                                 tag_sh.at[pl.ds(tbase + 16 * 4096, 8)], sem))
    for c in zcps:
      c.wait()
    for t in range(lu_per):
      ci = w + t * ns

      @pl.when(ci < n_lu)
      def _():
        pltpu.sync_copy(lu_hbm.at[pl.ds(ci * lu_chunk, lu_chunk)], lubuf)
        pltpu.sync_copy(lubuf, lu_out.at[pl.ds(ci * lu_chunk, lu_chunk)])

    plsc.subcore_barrier()

    def bit_round(t, carry):
      b = 13 - t
      gs = [pltpu.async_copy(tag_sh.at[uidx_v.at[j]], widx_v.at[j], sem)
            for j in range(nchunk)]
      for g in gs:
        g.wait()
      for j in range(nchunk):
        for k in range(8):
          sl = pl.ds(k * 16, 16)
          e = e_v[j, sl]
          cur = widx_v[j, sl]
          alive = jnp.logical_and((e >> (b + 1)) == (cur >> (b + 1)),
                                  ((e >> b) & 1) == 1)
          srcs_v[j, sl] = (e >> b) << b
          widx_v[j, sl] = jnp.where(alive, uidx_v[j, sl], dummy)
      ss = [pltpu.async_copy(srcs_v.at[j], tag_sh.at[widx_v.at[j]], sem)
            for j in range(nchunk)]
      for s in ss:
        s.wait()
      plsc.subcore_barrier()
      return carry

    lax.fori_loop(0, 14, bit_round, jnp.int32(0))

    gs = [pltpu.async_copy(tag_sh.at[uidx_v.at[j]], srcs_v.at[j], sem)
          for j in range(nchunk)]
    for g in gs:
      g.wait()
    ts_g = [pltpu.async_copy(ts_hbm.at[srcs_v.at[j]], tsb_v.at[j], sem2)
            for j in range(nchunk)]
    for g in ts_g:
      g.wait()
    ts_s = [pltpu.async_copy(tsb_v.at[j], lu_out.at[uidx_v.at[j]], sem2)
            for j in range(nchunk)]
    rows = [rows_a, rows_b]
    pend_w = [None, None]
    pend_r = [None, None]
    pend_r[0] = pltpu.async_copy(nm_hbm.at[srcs_v.at[0]], rows_a, sem)
    for j in range(nchunk):
      buf = j % 2
      pend_r[buf].wait()
      pend_w[buf] = pltpu.async_copy(rows[buf],
                                     memc_hbm.at[uidx_v.at[j]], sem3)
      if j + 1 < nchunk:
        nxt = (j + 1) % 2
        if pend_w[nxt] is not None:
          pend_w[nxt].wait()
        pend_r[nxt] = pltpu.async_copy(nm_hbm.at[srcs_v.at[j + 1]],
                                       rows[nxt], sem)
    for p in pend_w:
      if p is not None:
        p.wait()
    for s in ts_s:
      s.wait()

  return scatter


def kernel(user_indices, item_indices, timestamps, interaction_features,
           memory, last_update, item_table, W1, b1, W2, b2, W_ih, W_hh,
           b_ih, b_hh):
  batch = user_indices.shape[0]
  num_users, d = memory.shape
  num_items = item_table.shape[0]
  m = interaction_features.shape[1]

  uidx2 = user_indices.reshape(batch // 128, 128)
  iidx2 = item_indices.reshape(batch // 128, 128)

  user_mem, item_emb = _make_gather(num_users, num_items, d, batch)(
      memory, item_table, uidx2, iidx2)

  w1t = W1.T
  w1u, w1i, w1f = w1t[:d], w1t[d:2 * d], w1t[2 * d:]
  new_mem, mem_c = _make_dense(batch, d, m, num_users, 2048)(
      user_mem, item_emb, interaction_features,
      w1u, w1i, w1f, b1.reshape(1, m), W2.T, b2.reshape(1, m),
      W_ih.T, W_hh.T, b_ih.reshape(1, 3 * d), b_hh.reshape(1, 3 * d),
      memory)

  lu_out = _make_scatter(num_users, d, batch)(
      uidx2, new_mem, timestamps, last_update, mem_c)

  return new_mem, mem_c, lu_out


# speedup vs baseline: 4.9340x; 1.0593x over previous
import functools

import jax
import jax.numpy as jnp
from jax import lax
from jax.experimental import pallas as pl
from jax.experimental.pallas import tpu as pltpu
from jax.experimental.pallas import tpu_sc as plsc


@functools.lru_cache(maxsize=None)
def _make_gather(num_users, num_items, d, batch):
  nc, ns = 2, 16
  nw = nc * ns
  bpw = batch // nw
  nchunk = bpw // 128
  mesh = plsc.VectorSubcoreMesh(core_axis_name="c", subcore_axis_name="s")

  @functools.partial(
      pl.kernel,
      out_type=(
          jax.ShapeDtypeStruct((batch, d), jnp.float32),
          jax.ShapeDtypeStruct((batch, d), jnp.float32),
      ),
      mesh=mesh,
      scratch_types=[
          pltpu.VMEM((nchunk, 128), jnp.int32),
          pltpu.VMEM((nchunk, 128), jnp.int32),
          pltpu.VMEM((128, d), jnp.float32),
          pltpu.VMEM((128, d), jnp.float32),
          pltpu.SemaphoreType.DMA,
          pltpu.SemaphoreType.DMA,
      ],
  )
  def gather(mem_hbm, item_hbm, uidx_hbm, iidx_hbm, umem_out, iemb_out,
             uidx_v, iidx_v, rows_u, rows_i, sem_u, sem_i):
    c = lax.axis_index("c")
    s = lax.axis_index("s")
    wid = s * nc + c
    base = wid * nchunk
    pltpu.sync_copy(uidx_hbm.at[pl.ds(base, nchunk)], uidx_v)
    pltpu.sync_copy(iidx_hbm.at[pl.ds(base, nchunk)], iidx_v)
    for j in range(nchunk):
      r = (base + j) * 128
      cp_u = pltpu.async_copy(mem_hbm.at[uidx_v.at[j]], rows_u, sem_u)
      cp_i = pltpu.async_copy(item_hbm.at[iidx_v.at[j]], rows_i, sem_i)
      cp_u.wait()
      pltpu.sync_copy(rows_u, umem_out.at[pl.ds(r, 128)])
      cp_i.wait()
      pltpu.sync_copy(rows_i, iemb_out.at[pl.ds(r, 128)])

  return gather


@functools.lru_cache(maxsize=None)
def _make_dense(batch, d, m, num_users, bm):
  grid = (batch // bm,)

  def body(um, ie, ft, w1u, w1i, w1f, b1, w2, b2, wih, whh, bih, bhh,
           mem_in, nm_out, mem_out):
    del mem_in, mem_out
    h = jnp.dot(um[...], w1u[...], preferred_element_type=jnp.float32)
    h = h + jnp.dot(ie[...], w1i[...], preferred_element_type=jnp.float32)
    h = h + jnp.dot(ft[...], w1f[...], preferred_element_type=jnp.float32)
    h = jnp.maximum(h + b1[...], 0.0)
    msg = jnp.dot(h, w2[...], preferred_element_type=jnp.float32) + b2[...]
    gi = jnp.dot(msg, wih[...], preferred_element_type=jnp.float32) + bih[...]
    gh = jnp.dot(um[...], whh[...], preferred_element_type=jnp.float32) + bhh[...]
    r = jax.nn.sigmoid(gi[:, :d] + gh[:, :d])
    z = jax.nn.sigmoid(gi[:, d:2 * d] + gh[:, d:2 * d])
    n = jnp.tanh(gi[:, 2 * d:] + r * gh[:, 2 * d:])
    nm_out[...] = (1.0 - z) * n + z * um[...]

  bspec = lambda shape: pl.BlockSpec(shape, lambda i: (0, 0))
  return pl.pallas_call(
      body,
      grid=grid,
      in_specs=[
          pl.BlockSpec((bm, d), lambda i: (i, 0)),
          pl.BlockSpec((bm, d), lambda i: (i, 0)),
          pl.BlockSpec((bm, m), lambda i: (i, 0)),
          bspec((d, m)),
          bspec((d, m)),
          bspec((m, m)),
          bspec((1, m)),
          bspec((m, m)),
          bspec((1, m)),
          bspec((m, 3 * d)),
          bspec((d, 3 * d)),
          bspec((1, 3 * d)),
          bspec((1, 3 * d)),
          pl.BlockSpec(memory_space=pl.ANY),
      ],
      out_specs=[
          pl.BlockSpec((bm, d), lambda i: (i, 0)),
          pl.BlockSpec(memory_space=pl.ANY),
      ],
      out_shape=[
          jax.ShapeDtypeStruct((batch, d), jnp.float32),
          jax.ShapeDtypeStruct((num_users, d), jnp.float32),
      ],
      input_output_aliases={13: 1},
      compiler_params=pltpu.CompilerParams(
          dimension_semantics=("arbitrary",)),
  )


@functools.lru_cache(maxsize=None)
def _make_tags(num_users, d, batch):
  ns = 16
  bpw = batch // ns
  nchunk = bpw // 128
  lu_chunk = 20000
  n_lu = num_users // lu_chunk
  lu_per = (n_lu + ns - 1) // ns
  tag_size = 1048704
  tag_per = tag_size // ns
  dummy = num_users
  mesh = plsc.VectorSubcoreMesh(
      core_axis_name="c", subcore_axis_name="s", num_cores=1)

  @functools.partial(
      pl.kernel,
      out_type=(
          jax.ShapeDtypeStruct((batch // 128, 128), jnp.int32),
          jax.ShapeDtypeStruct((num_users,), jnp.float32),
      ),
      mesh=mesh,
      scratch_types=[
          pltpu.VMEM((nchunk, 128), jnp.int32),
          pltpu.VMEM((nchunk, 128), jnp.int32),
          pltpu.VMEM((nchunk, 128), jnp.int32),
          pltpu.VMEM((nchunk, 128), jnp.int32),
          pltpu.VMEM((4096,), jnp.int32),
          pltpu.VMEM((lu_chunk,), jnp.float32),
          pltpu.VMEM_SHARED((tag_size,), jnp.int32),
          pltpu.SemaphoreType.DMA,
          pltpu.SemaphoreType.DMA,
      ],
      compiler_params=pltpu.CompilerParams(has_side_effects=True),
  )
  def tags(uidx_hbm, lu_hbm, srcs_out, lu_out,
           uidx_v, e_v, widx_v, srcs_v, zer_v, lubuf, tag_sh, sem, sem2):
    w = lax.axis_index("s")
    base = w * nchunk
    pltpu.sync_copy(uidx_hbm.at[pl.ds(base, nchunk)], uidx_v)
    lanes = lax.iota(jnp.int32, 16)
    for j in range(nchunk):
      for k in range(8):
        e_v[j, pl.ds(k * 16, 16)] = w * bpw + j * 128 + k * 16 + lanes
    for k in range(256):
      zer_v[pl.ds(k * 16, 16)] = jnp.zeros((16,), jnp.int32)
    tbase = w * tag_per
    zcps = [pltpu.async_copy(zer_v, tag_sh.at[pl.ds(tbase + t * 4096, 4096)],
                             sem) for t in range(16)]
    zcps.append(pltpu.async_copy(zer_v.at[pl.ds(0, 8)],
                                 tag_sh.at[pl.ds(tbase + 16 * 4096, 8)], sem))
    for c in zcps:
      c.wait()
    for t in range(lu_per):
      ci = w + t * ns

      @pl.when(ci < n_lu)
      def _():
        pltpu.sync_copy(lu_hbm.at[pl.ds(ci * lu_chunk, lu_chunk)], lubuf)
        pltpu.sync_copy(lubuf, lu_out.at[pl.ds(ci * lu_chunk, lu_chunk)])

    plsc.subcore_barrier()

    def bit_round(t, carry):
      b = 13 - t
      gs = [pltpu.async_copy(tag_sh.at[uidx_v.at[j]], widx_v.at[j], sem)
            for j in range(nchunk)]
      for g in gs:
        g.wait()
      for j in range(nchunk):
        for k in range(8):
          sl = pl.ds(k * 16, 16)
          e = e_v[j, sl]
          cur = widx_v[j, sl]
          alive = jnp.logical_and((e >> (b + 1)) == (cur >> (b + 1)),
                                  ((e >> b) & 1) == 1)
          srcs_v[j, sl] = (e >> b) << b
          widx_v[j, sl] = jnp.where(alive, uidx_v[j, sl], dummy)
      ss = [pltpu.async_copy(srcs_v.at[j], tag_sh.at[widx_v.at[j]], sem)
            for j in range(nchunk)]
      for s in ss:
        s.wait()
      plsc.subcore_barrier()
      return carry

    lax.fori_loop(0, 14, bit_round, jnp.int32(0))

    gs = [pltpu.async_copy(tag_sh.at[uidx_v.at[j]], srcs_v.at[j], sem)
          for j in range(nchunk)]
    for g in gs:
      g.wait()
    pltpu.sync_copy(srcs_v, srcs_out.at[pl.ds(base, nchunk)])

  return tags


@functools.lru_cache(maxsize=None)
def _make_writeback(num_users, d, batch):
  ns = 16
  bpw = batch // ns
  nchunk = bpw // 128
  mesh = plsc.VectorSubcoreMesh(
      core_axis_name="c", subcore_axis_name="s", num_cores=1)

  @functools.partial(
      pl.kernel,
      out_type=jax.ShapeDtypeStruct((8, 128), jnp.int32),
      mesh=mesh,
      scratch_types=[
          pltpu.VMEM((nchunk, 128), jnp.int32),
          pltpu.VMEM((nchunk, 128), jnp.int32),
          pltpu.VMEM((128, d), jnp.float32),
          pltpu.VMEM((128, d), jnp.float32),
          pltpu.VMEM((nchunk, 128), jnp.float32),
          pltpu.SemaphoreType.DMA,
          pltpu.SemaphoreType.DMA,
          pltpu.SemaphoreType.DMA,
      ],
      compiler_params=pltpu.CompilerParams(has_side_effects=True),
  )
  def writeback(uidx_hbm, srcs_hbm, nm_hbm, ts_hbm, memc_hbm, luc_hbm, tok,
                uidx_v, srcs_v, rows_a, rows_b, tsb_v, sem, sem2, sem3):
    del tok
    w = lax.axis_index("s")
    base = w * nchunk
    pltpu.sync_copy(uidx_hbm.at[pl.ds(base, nchunk)], uidx_v)
    pltpu.sync_copy(srcs_hbm.at[pl.ds(base, nchunk)], srcs_v)
    ts_g = [pltpu.async_copy(ts_hbm.at[srcs_v.at[j]], tsb_v.at[j], sem2)
            for j in range(nchunk)]
    for g in ts_g:
      g.wait()
    ts_s = [pltpu.async_copy(tsb_v.at[j], luc_hbm.at[uidx_v.at[j]], sem2)
            for j in range(nchunk)]
    rows = [rows_a, rows_b]
    pend_w = [None, None]
    pend_r = [None, None]
    pend_r[0] = pltpu.async_copy(nm_hbm.at[srcs_v.at[0]], rows_a, sem)
    for j in range(nchunk):
      buf = j % 2
      pend_r[buf].wait()
      pend_w[buf] = pltpu.async_copy(rows[buf],
                                     memc_hbm.at[uidx_v.at[j]], sem3)
      if j + 1 < nchunk:
        nxt = (j + 1) % 2
        if pend_w[nxt] is not None:
          pend_w[nxt].wait()
        pend_r[nxt] = pltpu.async_copy(nm_hbm.at[srcs_v.at[j + 1]],
                                       rows[nxt], sem)
    for p in pend_w:
      if p is not None:
        p.wait()
    for s in ts_s:
      s.wait()

  return writeback


def kernel(user_indices, item_indices, timestamps, interaction_features,
           memory, last_update, item_table, W1, b1, W2, b2, W_ih, W_hh,
           b_ih, b_hh):
  batch = user_indices.shape[0]
  num_users, d = memory.shape
  num_items = item_table.shape[0]
  m = interaction_features.shape[1]

  uidx2 = user_indices.reshape(batch // 128, 128)
  iidx2 = item_indices.reshape(batch // 128, 128)

  user_mem, item_emb = _make_gather(num_users, num_items, d, batch)(
      memory, item_table, uidx2, iidx2)

  srcs2, lu_c = _make_tags(num_users, d, batch)(uidx2, last_update)

  w1t = W1.T
  w1u, w1i, w1f = w1t[:d], w1t[d:2 * d], w1t[2 * d:]
  new_mem, mem_c = _make_dense(batch, d, m, num_users, 2048)(
      user_mem, item_emb, interaction_features,
      w1u, w1i, w1f, b1.reshape(1, m), W2.T, b2.reshape(1, m),
      W_ih.T, W_hh.T, b_ih.reshape(1, 3 * d), b_hh.reshape(1, 3 * d),
      memory)

  tok = _make_writeback(num_users, d, batch)(
      uidx2, srcs2, new_mem, timestamps, mem_c, lu_c)

  mem_f, lu_f, _ = lax.optimization_barrier((mem_c, lu_c, tok))
  return new_mem, mem_f, lu_f
